# baseline (speedup 1.0000x reference)
import jax
import jax.numpy as jnp
from jax.experimental import pallas as pl

NUM_FRAMES = 16
HIDDEN_DIM = 768


def _body(in_ref, emb_ref, out_ref):
    for f in range(NUM_FRAMES):
        out_ref[:, f * HIDDEN_DIM:(f + 1) * HIDDEN_DIM] = (
            in_ref[f] + emb_ref[pl.ds(f, 1), :]
        )


def kernel(inputs, emb_table):
    P = inputs.shape[1]
    F, H = NUM_FRAMES, HIDDEN_DIM
    num_videos = inputs.shape[0] // F
    out = pl.pallas_call(
        _body,
        grid=(num_videos,),
        in_specs=[
            pl.BlockSpec((F, P, H), lambda b: (b, 0, 0)),
            pl.BlockSpec((F, H), lambda b: (0, 0)),
        ],
        out_specs=pl.BlockSpec((None, P, F * H), lambda b: (b, 0, 0)),
        out_shape=jax.ShapeDtypeStruct((num_videos, P, F * H), jnp.float32),
    )(inputs, emb_table)
    return out.reshape(num_videos * P, F, H)
